# baseline (speedup 1.0000x reference)
import functools

import jax
import jax.numpy as jnp
from jax import lax
from jax.experimental import pallas as pl
from jax.experimental.pallas import tpu as pltpu
from jax.experimental.pallas import tpu_sc as plsc

NC, NS, L = 2, 16, 16
NW = NC * NS
H = 128
T = 128
G = T // L
EPS = 1e-12


def _rsqrt(x):
    xi = lax.bitcast_convert_type(x, jnp.int32)
    yi = jnp.int32(0x5F3759DF) - lax.shift_right_arithmetic(xi, 1)
    y = lax.bitcast_convert_type(yi, jnp.float32)
    for _ in range(3):
        y = y * (1.5 - 0.5 * x * y * y)
    return y


def _body(n_chunks, wids_hbm, ptids_hbm, word_hbm, pos_hbm, type_hbm, gb_hbm,
          out_hbm, rows, widsb, ptidsb, posv, typv, gbv, slab,
          sem_g, sem_s, sem_i, sem_t):
    wid = lax.axis_index("s") * NC + lax.axis_index("c")
    lanes = lax.iota(jnp.int32, L)

    pltpu.async_copy(pos_hbm, posv, sem_t).wait()
    pltpu.async_copy(type_hbm, typv, sem_t).wait()
    pltpu.async_copy(gb_hbm, gbv, sem_t).wait()

    def start_ids(g, b):
        c = wid * n_chunks + g
        pltpu.async_copy(wids_hbm.at[c], widsb.at[b], sem_i[b])
        pltpu.async_copy(ptids_hbm.at[c], ptidsb.at[b], sem_i[b])

    def wait_ids(b):
        pltpu.make_async_copy(wids_hbm.at[0], widsb.at[b], sem_i[b]).wait()
        pltpu.make_async_copy(ptids_hbm.at[0], ptidsb.at[b], sem_i[b]).wait()

    def start_gather(b):
        pltpu.async_copy(word_hbm.at[widsb.at[b]], rows.at[b], sem_g[b])

    def compute_chunk(b):
        rowsb = rows.at[b]

        def group_body(g, _):
            tok = g * L + lanes
            pid = ptidsb[b, 0, g, :]
            tid = ptidsb[b, 1, g, :]

            def p1(d, carry):
                acc_s, acc_q = carry
                dfull = jnp.full((L,), d, jnp.int32)
                e = (plsc.load_gather(rowsb, [tok, dfull])
                     + plsc.load_gather(posv, [pid, dfull])
                     + plsc.load_gather(typv, [tid, dfull]))
                slab[d, :] = e
                return acc_s + e, acc_q + e * e

            acc_s, acc_q = lax.fori_loop(
                0, H, p1, (jnp.zeros((L,), jnp.float32),) * 2, unroll=4)
            mean = acc_s * (1.0 / H)
            var = acc_q * (1.0 / H) - mean * mean
            rstd = _rsqrt(var + EPS)

            def p2(d, _):
                dfull = jnp.full((L,), d, jnp.int32)
                gv = plsc.load_gather(gbv, [jnp.zeros((L,), jnp.int32), dfull])
                bv = plsc.load_gather(gbv, [jnp.ones((L,), jnp.int32), dfull])
                o = (slab[d, :] - mean) * (rstd * gv) + bv
                plsc.store_scatter(rowsb, [tok, dfull], o)
                return 0

            lax.fori_loop(0, H, p2, 0, unroll=4)
            return 0

        lax.fori_loop(0, G, group_body, 0)

    def start_scatter(g, b):
        off = (wid * n_chunks + g) * T
        pltpu.async_copy(rows.at[b], out_hbm.at[pl.ds(off, T)], sem_s[b])

    def wait_scatter(b):
        pltpu.make_async_copy(
            rows.at[b], out_hbm.at[pl.ds(0, T)], sem_s[b]).wait()

    start_ids(0, 0)
    start_ids(1, 1)
    wait_ids(0)
    start_gather(0)

    def loop(g2, _):
        for par in range(2):
            g = g2 * 2 + par
            b = par
            nb = 1 - par
            @pl.when(g + 1 < n_chunks)
            def _():
                @pl.when(g - 1 >= 0)
                def _():
                    wait_scatter(nb)
                wait_ids(nb)
                start_gather(nb)

            pltpu.make_async_copy(
                word_hbm.at[widsb.at[b]], rows.at[b], sem_g[b]).wait()
            compute_chunk(b)
            start_scatter(g, b)

            @pl.when(g + 2 < n_chunks)
            def _():
                start_ids(g + 2, b)
        return 0

    lax.fori_loop(0, n_chunks // 2, loop, 0)
    for b in range(2):
        wait_scatter(b)


def kernel(input_ids, position_ids, token_type_ids, word_emb, pos_emb,
           type_emb, ln_gamma, ln_beta):
    B, Lseq = input_ids.shape
    NT = B * Lseq
    n_chunks = NT // (NW * T)
    C = NW * n_chunks

    wids = input_ids.reshape(C, T).astype(jnp.int32)
    ptids = jnp.stack([
        position_ids.reshape(C, G, L).astype(jnp.int32),
        token_type_ids.reshape(C, G, L).astype(jnp.int32),
    ], axis=1)
    gb = jnp.stack([ln_gamma, ln_beta], axis=0)

    mesh = plsc.VectorSubcoreMesh(
        core_axis_name="c", subcore_axis_name="s",
        num_cores=NC, num_subcores=NS)
    run = pl.kernel(
        functools.partial(_body, n_chunks),
        out_type=jax.ShapeDtypeStruct((NT, H), jnp.float32),
        mesh=mesh,
        compiler_params=pltpu.CompilerParams(needs_layout_passes=False),
        scratch_types=[
            pltpu.VMEM((2, T, H), jnp.float32),
            pltpu.VMEM((2, T), jnp.int32),
            pltpu.VMEM((2, 2, G, L), jnp.int32),
            pltpu.VMEM((512, H), jnp.float32),
            pltpu.VMEM((2, H), jnp.float32),
            pltpu.VMEM((2, H), jnp.float32),
            pltpu.VMEM((H, L), jnp.float32),
            [pltpu.SemaphoreType.DMA] * 2,
            [pltpu.SemaphoreType.DMA] * 2,
            [pltpu.SemaphoreType.DMA] * 2,
            pltpu.SemaphoreType.DMA,
        ],
    )
    out = run(wids, ptids, word_emb, pos_emb, type_emb, gb)
    return out.reshape(B, Lseq, H)


# speedup vs baseline: 6.3415x; 6.3415x over previous
import functools

import jax
import jax.numpy as jnp
from jax import lax
from jax.experimental import pallas as pl
from jax.experimental.pallas import tpu as pltpu
from jax.experimental.pallas import tpu_sc as plsc

NC, NS, L = 2, 16, 16
NW = NC * NS
H = 128
J = H // L
T = 128
PT = 512 * 2
EPS = 1e-12


def _rsqrt(x):
    xi = lax.bitcast_convert_type(x, jnp.int32)
    yi = jnp.int32(0x5F3759DF) - lax.shift_right_arithmetic(xi, 1)
    y = lax.bitcast_convert_type(yi, jnp.float32)
    for _ in range(3):
        y = y * (1.5 - 0.5 * x * y * y)
    return y


def _body(n_chunks, wids_hbm, ptids_hbm, word_hbm, pt_hbm, gb_hbm, out_hbm,
          ptshared, rows, ptrows, widsb, ptidsb, gbv,
          sem_g, sem_p, sem_s, sem_i, sem_t):
    wid = lax.axis_index("s") * NC + lax.axis_index("c")

    @pl.when(lax.axis_index("s") == 0)
    def _():
        pltpu.async_copy(pt_hbm, ptshared, sem_t).wait()
    pltpu.async_copy(gb_hbm, gbv, sem_t).wait()
    plsc.subcore_barrier()

    def start_ids(g, b):
        c = wid * n_chunks + g
        pltpu.async_copy(wids_hbm.at[c], widsb.at[b], sem_i[b])
        pltpu.async_copy(ptids_hbm.at[c], ptidsb.at[b], sem_i[b])

    def wait_ids(b):
        pltpu.make_async_copy(wids_hbm.at[0], widsb.at[b], sem_i[b]).wait()
        pltpu.make_async_copy(ptids_hbm.at[0], ptidsb.at[b], sem_i[b]).wait()

    def start_gathers(b):
        pltpu.async_copy(word_hbm.at[widsb.at[b]], rows.at[b], sem_g[b])
        pltpu.async_copy(ptshared.at[ptidsb.at[b]], ptrows.at[b], sem_p[b])

    def wait_gathers(b):
        pltpu.make_async_copy(
            word_hbm.at[widsb.at[b]], rows.at[b], sem_g[b]).wait()
        pltpu.make_async_copy(
            ptshared.at[ptidsb.at[b]], ptrows.at[b], sem_p[b]).wait()

    def compute_chunk(b):
        rowsb = rows.at[b]
        ptrowsb = ptrows.at[b]
        gs = [gbv[0, j, :] for j in range(J)]
        bs = [gbv[1, j, :] for j in range(J)]
        inv_h = jnp.full((L,), 1.0 / H, jnp.float32)

        def tok(t, _):
            e = [rowsb[t, j, :] + ptrowsb[t, j, :] for j in range(J)]
            s01, s23 = e[0] + e[1], e[2] + e[3]
            s45, s67 = e[4] + e[5], e[6] + e[7]
            s = (s01 + s23) + (s45 + s67)
            q = [ej * ej for ej in e]
            q01, q23 = q[0] + q[1], q[2] + q[3]
            q45, q67 = q[4] + q[5], q[6] + q[7]
            qs = (q01 + q23) + (q45 + q67)
            tot = jnp.sum(s, axis=0)
            qtot = jnp.sum(qs, axis=0)
            mv = inv_h * tot
            var = inv_h * qtot - mv * mv
            rstd = _rsqrt(var + EPS)
            for j in range(J):
                rowsb[t, j, :] = (e[j] - mv) * (rstd * gs[j]) + bs[j]
            return 0

        lax.fori_loop(0, T, tok, 0, unroll=2)

    def start_scatter(g, b):
        off = (wid * n_chunks + g) * T
        pltpu.async_copy(rows.at[b], out_hbm.at[pl.ds(off, T)], sem_s[b])

    def wait_scatter(b):
        pltpu.make_async_copy(
            rows.at[b], out_hbm.at[pl.ds(0, T)], sem_s[b]).wait()

    start_ids(0, 0)
    start_ids(1, 1)
    wait_ids(0)
    start_gathers(0)

    def loop(g2, _):
        for par in range(2):
            g = g2 * 2 + par
            b = par
            nb = 1 - par
            @pl.when(g + 1 < n_chunks)
            def _():
                @pl.when(g - 1 >= 0)
                def _():
                    wait_scatter(nb)
                wait_ids(nb)
                start_gathers(nb)

            wait_gathers(b)
            compute_chunk(b)
            start_scatter(g, b)

            @pl.when(g + 2 < n_chunks)
            def _():
                start_ids(g + 2, b)
        return 0

    lax.fori_loop(0, n_chunks // 2, loop, 0)
    for b in range(2):
        wait_scatter(b)


def kernel(input_ids, position_ids, token_type_ids, word_emb, pos_emb,
           type_emb, ln_gamma, ln_beta):
    B, Lseq = input_ids.shape
    NT = B * Lseq
    n_chunks = NT // (NW * T)
    C = NW * n_chunks

    wids = input_ids.reshape(C, T).astype(jnp.int32)
    ptids = (position_ids.astype(jnp.int32) * 2
             + token_type_ids.astype(jnp.int32)).reshape(C, T)
    pt_table = (pos_emb[:, None, :] + type_emb[None, :, :]).reshape(
        PT, J, L)
    gb = jnp.stack([ln_gamma, ln_beta], axis=0).reshape(2, J, L)

    mesh = plsc.VectorSubcoreMesh(
        core_axis_name="c", subcore_axis_name="s",
        num_cores=NC, num_subcores=NS)
    run = pl.kernel(
        functools.partial(_body, n_chunks),
        out_type=jax.ShapeDtypeStruct((NT, J, L), jnp.float32),
        mesh=mesh,
        compiler_params=pltpu.CompilerParams(
            needs_layout_passes=False, use_tc_tiling_on_sc=False),
        scratch_types=[
            pltpu.VMEM_SHARED((PT, J, L), jnp.float32),
            pltpu.VMEM((2, T, J, L), jnp.float32),
            pltpu.VMEM((2, T, J, L), jnp.float32),
            pltpu.VMEM((2, T), jnp.int32),
            pltpu.VMEM((2, T), jnp.int32),
            pltpu.VMEM((2, J, L), jnp.float32),
            [pltpu.SemaphoreType.DMA] * 2,
            [pltpu.SemaphoreType.DMA] * 2,
            [pltpu.SemaphoreType.DMA] * 2,
            [pltpu.SemaphoreType.DMA] * 2,
            pltpu.SemaphoreType.DMA,
        ],
    )
    out = run(wids, ptids, word_emb.reshape(-1, J, L), pt_table, gb)
    return out.reshape(B, Lseq, H)


# speedup vs baseline: 11.7243x; 1.8488x over previous
import functools

import jax
import jax.numpy as jnp
from jax import lax
from jax.experimental import pallas as pl
from jax.experimental.pallas import tpu as pltpu
from jax.experimental.pallas import tpu_sc as plsc

NC, NS, L = 2, 16, 16
NW = NC * NS
H = 128
J = H // L
T = 128
PT = 512 * 2
TP = T + 1
R = 4
EPS = 1e-12


def _rsqrt(x):
    xi = lax.bitcast_convert_type(x, jnp.int32)
    yi = jnp.int32(0x5F3759DF) - lax.shift_right_arithmetic(xi, 1)
    y = lax.bitcast_convert_type(yi, jnp.float32)
    for _ in range(2):
        y = y * (1.5 - 0.5 * x * y * y)
    return y


def _body(n_chunks, wids_hbm, ptids_hbm, word_hbm, pt_hbm, gb_hbm, out_hbm,
          ptshared, rows, widsb, ptidsb, gbv, slab,
          sem_w, sem_a, sem_s, sem_i, sem_t):
    wid = lax.axis_index("s") * NC + lax.axis_index("c")

    @pl.when(lax.axis_index("s") == 0)
    def _():
        pltpu.async_copy(pt_hbm, ptshared, sem_t).wait()
    pltpu.async_copy(gb_hbm, gbv, sem_t).wait()
    plsc.subcore_barrier()

    def start_ids(g, r):
        c = wid * n_chunks + g
        pltpu.async_copy(wids_hbm.at[c], widsb.at[r], sem_i[r])
        pltpu.async_copy(ptids_hbm.at[c], ptidsb.at[r], sem_i[r])

    def wait_ids(r):
        pltpu.make_async_copy(wids_hbm.at[0], widsb.at[r], sem_i[r]).wait()
        pltpu.make_async_copy(ptids_hbm.at[0], ptidsb.at[r], sem_i[r]).wait()

    def start_word(r):
        pltpu.async_copy(word_hbm.at[widsb.at[r]], rows.at[r], sem_w[r])

    def wait_word(r):
        pltpu.make_async_copy(
            word_hbm.at[widsb.at[r]], rows.at[r], sem_w[r]).wait()

    def start_ptadd(r):
        pltpu.async_copy(
            ptshared.at[ptidsb.at[r]], rows.at[r], sem_a[r], add=True)

    def wait_ptadd(r):
        pltpu.make_async_copy(
            ptshared.at[ptidsb.at[r]], rows.at[r], sem_a[r]).wait()

    def compute_chunk(r):
        rowsb = rows.at[r]
        gs = [gbv[0, pl.ds(L * j, L)] for j in range(J)]
        bs = [gbv[1, pl.ds(L * j, L)] for j in range(J)]
        lanes = lax.iota(jnp.int32, L)
        fj = [(L * j + lanes) * TP for j in range(J)]

        @plsc.parallel_loop(0, T, 1, unroll=4)
        def _(t):
            tb = jnp.full((L,), t, jnp.int32)
            e = [rowsb[t, pl.ds(L * j, L)] for j in range(J)]
            for j in range(J):
                plsc.store_scatter(slab, [fj[j] + tb], e[j])

        @plsc.parallel_loop(0, T // L, 1)
        def _(g):
            def dpair(i, carry):
                s0, q0, s1, q1, off = carry
                v0 = slab[pl.ds(off, L)]
                v1 = slab[pl.ds(off + TP, L)]
                return (s0 + v0, q0 + v0 * v0, s1 + v1, q1 + v1 * v1,
                        off + 2 * TP)

            z = jnp.zeros((L,), jnp.float32)
            s0, q0, s1, q1, _o = lax.fori_loop(
                0, H // 2, dpair, (z, z, z, z, g * L), unroll=4)
            mv = (1.0 / H) * (s0 + s1)
            var = (1.0 / H) * (q0 + q1) - mv * mv
            rstd = _rsqrt(var + EPS)
            for k in range(L):
                t = g * L + k
                bm = mv[k]
                br = rstd[k]
                for j in range(J):
                    rowsb[t, pl.ds(L * j, L)] = (
                        rowsb[t, pl.ds(L * j, L)] - bm) * (br * gs[j]) + bs[j]

    def start_scatter(g, r):
        off = (wid * n_chunks + g) * T
        pltpu.async_copy(rows.at[r], out_hbm.at[pl.ds(off, T)], sem_s[r])

    def wait_scatter(r):
        pltpu.make_async_copy(
            rows.at[r], out_hbm.at[pl.ds(0, T)], sem_s[r]).wait()

    start_ids(0, 0)
    start_ids(1, 1)
    start_ids(2, 2)
    wait_ids(0)
    start_word(0)
    wait_word(0)
    start_ptadd(0)
    wait_ids(1)
    start_word(1)

    def loop(g4, _):
        for par in range(R):
            g = g4 * R + par
            r = par
            r1 = (par + 1) % R
            r2 = (par + 2) % R
            r3 = (par + 3) % R

            wait_ptadd(r)

            @pl.when(g + 1 < n_chunks)
            def _():
                wait_word(r1)
                start_ptadd(r1)

            compute_chunk(r)
            start_scatter(g, r)

            @pl.when(g + 2 < n_chunks)
            def _():
                @pl.when(g - 2 >= 0)
                def _():
                    wait_scatter(r2)
                wait_ids(r2)
                start_word(r2)

            @pl.when(g + 3 < n_chunks)
            def _():
                start_ids(g + 3, r3)
        return 0

    lax.fori_loop(0, n_chunks // R, loop, 0)
    for r in range(R):
        wait_scatter(r)


def kernel(input_ids, position_ids, token_type_ids, word_emb, pos_emb,
           type_emb, ln_gamma, ln_beta):
    B, Lseq = input_ids.shape
    NT = B * Lseq
    n_chunks = NT // (NW * T)
    C = NW * n_chunks

    wids = input_ids.reshape(C, T).astype(jnp.int32)
    ptids = (position_ids.astype(jnp.int32) * 2
             + token_type_ids.astype(jnp.int32)).reshape(C, T)
    pt_table = (pos_emb[:, None, :] + type_emb[None, :, :]).reshape(PT, H)
    gb = jnp.stack([ln_gamma, ln_beta], axis=0)

    mesh = plsc.VectorSubcoreMesh(
        core_axis_name="c", subcore_axis_name="s",
        num_cores=NC, num_subcores=NS)
    run = pl.kernel(
        functools.partial(_body, n_chunks),
        out_type=jax.ShapeDtypeStruct((NT, H), jnp.float32),
        mesh=mesh,
        compiler_params=pltpu.CompilerParams(needs_layout_passes=False),
        scratch_types=[
            pltpu.VMEM_SHARED((PT, H), jnp.float32),
            pltpu.VMEM((R, T, H), jnp.float32),
            pltpu.VMEM((R, T), jnp.int32),
            pltpu.VMEM((R, T), jnp.int32),
            pltpu.VMEM((2, H), jnp.float32),
            pltpu.VMEM((H * TP,), jnp.float32),
            [pltpu.SemaphoreType.DMA] * R,
            [pltpu.SemaphoreType.DMA] * R,
            [pltpu.SemaphoreType.DMA] * R,
            [pltpu.SemaphoreType.DMA] * R,
            pltpu.SemaphoreType.DMA,
        ],
    )
    out = run(wids, ptids, word_emb, pt_table, gb)
    return out.reshape(B, Lseq, H)


# speedup vs baseline: 12.7816x; 1.0902x over previous
import functools

import jax
import jax.numpy as jnp
from jax import lax
from jax.experimental import pallas as pl
from jax.experimental.pallas import tpu as pltpu
from jax.experimental.pallas import tpu_sc as plsc

NC, NS, L = 2, 16, 16
NW = NC * NS
H = 128
J = H // L
T = 128
PT = 512 * 2
TP = T + 1
EPS = 1e-12


def _rsqrt(x):
    xi = lax.bitcast_convert_type(x, jnp.int32)
    yi = jnp.int32(0x5F3759DF) - lax.shift_right_arithmetic(xi, 1)
    y = lax.bitcast_convert_type(yi, jnp.float32)
    for _ in range(2):
        y = y * (1.5 - 0.5 * x * y * y)
    return y


def _body(n_chunks, wids_hbm, ptids_hbm, word_hbm, pt_hbm, gb_hbm, out_hbm,
          ptshared, rows, ptrows, widsb, ptidsb, gbv, slab, stats,
          sem_g, sem_p, sem_s, sem_i, sem_t):
    wid = lax.axis_index("s") * NC + lax.axis_index("c")

    @pl.when(lax.axis_index("s") == 0)
    def _():
        pltpu.async_copy(pt_hbm, ptshared, sem_t).wait()
    pltpu.async_copy(gb_hbm, gbv, sem_t).wait()
    plsc.subcore_barrier()

    def start_ids(g, b):
        c = wid * n_chunks + g
        pltpu.async_copy(wids_hbm.at[c], widsb.at[b], sem_i[b])
        pltpu.async_copy(ptids_hbm.at[c], ptidsb.at[b], sem_i[b])

    def wait_ids(b):
        pltpu.make_async_copy(wids_hbm.at[0], widsb.at[b], sem_i[b]).wait()
        pltpu.make_async_copy(ptids_hbm.at[0], ptidsb.at[b], sem_i[b]).wait()

    def start_gathers(b):
        pltpu.async_copy(word_hbm.at[widsb.at[b]], rows.at[b], sem_g[b])
        pltpu.async_copy(ptshared.at[ptidsb.at[b]], ptrows.at[b], sem_p[b])

    def wait_gathers(b):
        pltpu.make_async_copy(
            word_hbm.at[widsb.at[b]], rows.at[b], sem_g[b]).wait()
        pltpu.make_async_copy(
            ptshared.at[ptidsb.at[b]], ptrows.at[b], sem_p[b]).wait()

    def compute_chunk(b):
        rowsb = rows.at[b]
        ptrowsb = ptrows.at[b]
        gs = [gbv[0, pl.ds(L * j, L)] for j in range(J)]
        bs = [gbv[1, pl.ds(L * j, L)] for j in range(J)]
        lanes = lax.iota(jnp.int32, L)
        fj = [(L * j + lanes) * TP for j in range(J)]

        @plsc.parallel_loop(0, T, 1, unroll=4)
        def _(t):
            tb = jnp.full((L,), t, jnp.int32)
            e = [rowsb[t, pl.ds(L * j, L)] + ptrowsb[t, pl.ds(L * j, L)]
                 for j in range(J)]
            for j in range(J):
                rowsb[t, pl.ds(L * j, L)] = e[j]
                plsc.store_scatter(slab, [fj[j] + tb], e[j])

        @plsc.parallel_loop(0, T // L, 1)
        def _(g):
            def dpair(i, carry):
                s0, q0, s1, q1, off = carry
                v0 = slab[pl.ds(off, L)]
                v1 = slab[pl.ds(off + TP, L)]
                return (s0 + v0, q0 + v0 * v0, s1 + v1, q1 + v1 * v1,
                        off + 2 * TP)

            z = jnp.zeros((L,), jnp.float32)
            s0, q0, s1, q1, _o = lax.fori_loop(
                0, H // 2, dpair, (z, z, z, z, g * L), unroll=4)
            mv = (1.0 / H) * (s0 + s1)
            var = (1.0 / H) * (q0 + q1) - mv * mv
            rstd = _rsqrt(var + EPS)
            for k in range(L):
                stats[0, g * L + k, :] = jnp.full((L,), mv[k], jnp.float32)
                stats[1, g * L + k, :] = jnp.full((L,), rstd[k], jnp.float32)

        @plsc.parallel_loop(0, T, 1, unroll=2)
        def _(t):
            bm = stats[0, t, :]
            br = stats[1, t, :]
            e = [rowsb[t, pl.ds(L * j, L)] for j in range(J)]
            for j in range(J):
                rowsb[t, pl.ds(L * j, L)] = (e[j] - bm) * (br * gs[j]) + bs[j]

    def start_scatter(g, b):
        off = (wid * n_chunks + g) * T
        pltpu.async_copy(rows.at[b], out_hbm.at[pl.ds(off, T)], sem_s[b])

    def wait_scatter(b):
        pltpu.make_async_copy(
            rows.at[b], out_hbm.at[pl.ds(0, T)], sem_s[b]).wait()

    start_ids(0, 0)
    start_ids(1, 1)
    wait_ids(0)
    start_gathers(0)

    def loop(g2, _):
        for par in range(2):
            g = g2 * 2 + par
            b = par
            nb = 1 - par
            @pl.when(g + 1 < n_chunks)
            def _():
                @pl.when(g - 1 >= 0)
                def _():
                    wait_scatter(nb)
                wait_ids(nb)
                start_gathers(nb)

            wait_gathers(b)
            compute_chunk(b)
            start_scatter(g, b)

            @pl.when(g + 2 < n_chunks)
            def _():
                start_ids(g + 2, b)
        return 0

    lax.fori_loop(0, n_chunks // 2, loop, 0)
    for b in range(2):
        wait_scatter(b)


def kernel(input_ids, position_ids, token_type_ids, word_emb, pos_emb,
           type_emb, ln_gamma, ln_beta):
    B, Lseq = input_ids.shape
    NT = B * Lseq
    n_chunks = NT // (NW * T)
    C = NW * n_chunks

    wids = input_ids.reshape(C, T).astype(jnp.int32)
    ptids = (position_ids.astype(jnp.int32) * 2
             + token_type_ids.astype(jnp.int32)).reshape(C, T)
    pt_table = (pos_emb[:, None, :] + type_emb[None, :, :]).reshape(PT, H)
    gb = jnp.stack([ln_gamma, ln_beta], axis=0)

    mesh = plsc.VectorSubcoreMesh(
        core_axis_name="c", subcore_axis_name="s",
        num_cores=NC, num_subcores=NS)
    run = pl.kernel(
        functools.partial(_body, n_chunks),
        out_type=jax.ShapeDtypeStruct((NT, H), jnp.float32),
        mesh=mesh,
        compiler_params=pltpu.CompilerParams(needs_layout_passes=False),
        scratch_types=[
            pltpu.VMEM_SHARED((PT, H), jnp.float32),
            pltpu.VMEM((2, T, H), jnp.float32),
            pltpu.VMEM((2, T, H), jnp.float32),
            pltpu.VMEM((2, T), jnp.int32),
            pltpu.VMEM((2, T), jnp.int32),
            pltpu.VMEM((2, H), jnp.float32),
            pltpu.VMEM((H * TP,), jnp.float32),
            pltpu.VMEM((2, T, L), jnp.float32),
            [pltpu.SemaphoreType.DMA] * 2,
            [pltpu.SemaphoreType.DMA] * 2,
            [pltpu.SemaphoreType.DMA] * 2,
            [pltpu.SemaphoreType.DMA] * 2,
            pltpu.SemaphoreType.DMA,
        ],
    )
    out = run(wids, ptids, word_emb, pt_table, gb)
    return out.reshape(B, Lseq, H)
